# baseline (speedup 1.0000x reference)
import functools

import jax
import jax.numpy as jnp
from jax import lax
from jax.experimental import pallas as pl
from jax.experimental.pallas import tpu as pltpu
from jax.experimental.pallas import tpu_sc as plsc

N = 10000
E = 160000
NT = 32
TB = 320
NPAD = 10240
CE = 2000
FB = 2048
CAPE = FB * 80
BE = 128

_mesh = plsc.VectorSubcoreMesh(core_axis_name="c", subcore_axis_name="s")
_sc_params = pltpu.CompilerParams(needs_layout_passes=False)

f32 = jnp.float32
i32 = jnp.int32


def _wid():
    return lax.axis_index("s") * 2 + lax.axis_index("c")


def _iota16():
    return lax.iota(i32, 16)


def _compact_body(src_g, dst_g, src_o, doff_o, cnt_o, s_ch, d_ch, stg_s, stg_d, cntv):
    t = _wid()
    lo = t * TB
    hi = jnp.minimum(lo + TB, N)

    def append_vreg(cur, sv, dv, m):
        ranks = plsc.cumsum(m.astype(i32))
        idx = cur + ranks - 1
        plsc.store_scatter(stg_s, [idx], sv, mask=m)
        plsc.store_scatter(stg_d, [idx], dv - lo, mask=m)
        return cur + ranks[15]

    def maybe_flush(carry):
        cur, nf = carry

        def do_flush(c):
            foff = pl.multiple_of(t * CAPE + nf * FB, 8)
            pltpu.sync_copy(stg_s.at[pl.ds(0, FB)],
                            src_o.at[pl.ds(foff, FB)])
            pltpu.sync_copy(stg_d.at[pl.ds(0, FB)],
                            doff_o.at[pl.ds(foff, FB)])
            nmv = c - FB

            def mv(j, _):
                v = stg_s[pl.ds(FB + j * 16, 16)]
                stg_s[pl.ds(j * 16, 16)] = v
                v2 = stg_d[pl.ds(FB + j * 16, 16)]
                stg_d[pl.ds(j * 16, 16)] = v2
                return 0

            lax.fori_loop(0, (nmv + 15) // 16, mv, 0)
            return (c - FB, nf + 1)

        return lax.cond(cur >= FB, do_flush, lambda c: (c, nf), cur)

    def chunk_body(k, carry):
        cur, nf = carry
        off = pl.multiple_of(k * CE, 8)
        pltpu.sync_copy(src_g.at[pl.ds(off, CE)], s_ch)
        pltpu.sync_copy(dst_g.at[pl.ds(off, CE)], d_ch)

        def vec_body(i, c):
            sv = s_ch[pl.ds(i * 16, 16)]
            dv = d_ch[pl.ds(i * 16, 16)]
            m = (dv >= lo) & (dv < lo + TB)
            return append_vreg(c, sv, dv, m)

        cur = lax.fori_loop(0, CE // 16, vec_body, cur)
        return maybe_flush((cur, nf))

    cur, nf = lax.fori_loop(0, E // CE, chunk_body, (0, 0))

    def self_body(i, carry):
        c, f = carry
        dg = lo + i * 16 + _iota16()
        m = dg < hi
        c = append_vreg(c, dg, dg, m)
        return maybe_flush((c, f))

    cur, nf = lax.fori_loop(0, TB // 16, self_body, (cur, nf))
    total = nf * FB + cur
    foff = pl.multiple_of(t * CAPE + nf * FB, 8)
    pltpu.sync_copy(stg_s.at[pl.ds(0, FB)], src_o.at[pl.ds(foff, FB)])
    pltpu.sync_copy(stg_d.at[pl.ds(0, FB)], doff_o.at[pl.ds(foff, FB)])
    cntv[...] = jnp.broadcast_to(total, (16,)).astype(i32)
    pltpu.sync_copy(cntv, cnt_o.at[pl.ds(t * 16, 16)])


_compact = functools.partial(
    pl.kernel,
    mesh=_mesh,
    out_type=[
        jax.ShapeDtypeStruct((NT * CAPE,), i32),
        jax.ShapeDtypeStruct((NT * CAPE,), i32),
        jax.ShapeDtypeStruct((NT * 16,), i32),
    ],
    scratch_types=[
        pltpu.VMEM((CE,), i32),
        pltpu.VMEM((CE,), i32),
        pltpu.VMEM((2 * FB,), i32),
        pltpu.VMEM((2 * FB,), i32),
        pltpu.VMEM((16,), i32),
    ],
    compiler_params=_sc_params,
)(_compact_body)


def _make_gat_sc(Cc, H):
    def body(hc, asad, src_l, doff_l, cnt_i, out, w_l,
             rdenom, sstg, dstg, wstg, cntv, sem, sem2):
        t = _wid()
        lo = t * TB
        ebase = t * CAPE
        pltpu.sync_copy(cnt_i, cntv)
        cnt = cntv[pl.ds(t * 16, 16)][0]
        nblk = (cnt + FB - 1) // FB

        def phase1(asad_t, denom):
            pltpu.sync_copy(asad, asad_t)

            def z(i, _):
                denom[pl.ds(i * 16, 16)] = jnp.zeros((16,), f32)
                return 0

            lax.fori_loop(0, TB * H, z, 0)

            def blk_body(blk, _):
                boff = pl.multiple_of(ebase + blk * FB, 8)
                pltpu.sync_copy(src_l.at[pl.ds(boff, FB)], sstg)
                pltpu.sync_copy(doff_l.at[pl.ds(boff, FB)], dstg)

                def vec_body(i, _):
                    e0 = blk * FB + i * 16
                    m = (e0 + _iota16()) < cnt
                    sv = jnp.where(m, sstg[pl.ds(i * 16, 16)], 0)
                    dv = jnp.where(m, dstg[pl.ds(i * 16, 16)], 0)
                    dg = dv + lo
                    for h in range(H):
                        a1 = plsc.load_gather(asad_t, [sv * 8 + h])
                        a2 = plsc.load_gather(asad_t, [dg * 8 + 4 + h])
                        al = a1 + a2
                        al = jnp.maximum(al, 0.2 * al)
                        w = jnp.exp(al)
                        wstg[pl.ds(h * FB + i * 16, 16)] = w
                        plsc.addupdate_scatter(
                            denom, [_iota16() * (TB * H) + dv * H + h], w, mask=m)
                    return 0

                lax.fori_loop(0, FB // 16, vec_body, 0)
                for h in range(H):
                    woff = pl.multiple_of(t * (H * CAPE) + h * CAPE + blk * FB, 8)
                    pltpu.sync_copy(wstg.at[pl.ds(h * FB, FB)], w_l.at[pl.ds(woff, FB)])
                return 0

            lax.fori_loop(0, nblk, blk_body, 0)

            def rd(j, _):
                s = jnp.zeros((16,), f32)
                for st in range(16):
                    s = s + denom[pl.ds(st * (TB * H) + j * 16, 16)]
                rdenom[pl.ds(j * 16, 16)] = 1.0 / (s + 1e-16)
                return 0

            lax.fori_loop(0, (TB * H) // 16, rd, 0)

        pl.run_scoped(phase1,
                      pltpu.VMEM((NPAD * 8,), f32),
                      pltpu.VMEM((16 * TB * H,), f32))

        NSUB = FB // BE

        def phase2(acc, wb2f, idxb, rows):
            sems = (sem, sem2)

            def gslice(par):
                return (hc.at[idxb], rows)

            def chunk_body(c, _):
                head = c >> 2

                def zacc(j, _):
                    acc[j >> 3, pl.ds((j & 7) * 16, 16)] = jnp.zeros((16,), f32)
                    return 0

                lax.fori_loop(0, TB * 8, zacc, 0)

                def mkidx(blk, g, par):
                    def mi(i, _):
                        el = g * BE + i * 16
                        m = (blk * FB + el + _iota16()) < cnt
                        sv = jnp.where(m, sstg[pl.ds(el, 16)], 0)
                        idxb[pl.ds(i * 16, 16)] = sv + c * NPAD
                        return 0

                    lax.fori_loop(0, BE // 16, mi, 0)
                    s, d = gslice(par)
                    pltpu.async_copy(s, d, sems[par]).wait()

                def process(blk, g, par):

                    def edge_vec_body(i, _):
                        el = g * BE + i * 16
                        m = (blk * FB + el + _iota16()) < cnt
                        do16 = jnp.where(m, dstg[pl.ds(el, 16)], 0)
                        w16 = wb2f[pl.ds(el, 16)]
                        r16 = plsc.load_gather(rdenom, [do16 * H + head])
                        att16 = jnp.where(m, w16 * r16, 0.0)
                        for jj in range(16):
                            att = att16[jj]
                            do = do16[jj]
                            for k in range(8):
                                v = rows[i * 16 + jj, pl.ds(k * 16, 16)] * att
                                plsc.addupdate(acc.at[do, pl.ds(k * 16, 16)], v)
                        return 0

                    lax.fori_loop(0, BE // 16, edge_vec_body, 0)

                def blk2(blk, _):
                    boff = pl.multiple_of(ebase + blk * FB, 8)
                    pltpu.sync_copy(src_l.at[pl.ds(boff, FB)], sstg)
                    pltpu.sync_copy(doff_l.at[pl.ds(boff, FB)], dstg)
                    woff = pl.multiple_of(
                        t * (H * CAPE) + head * CAPE + blk * FB, 8)
                    pltpu.sync_copy(w_l.at[pl.ds(woff, FB)], wb2f)
                    def gp_body(g, _):
                        mkidx(blk, g, 0)
                        process(blk, g, 0)
                        return 0

                    lax.fori_loop(0, NSUB, gp_body, 0)
                    return 0

                lax.fori_loop(0, (cnt + FB - 1) // FB, blk2, 0)
                ooff = pl.multiple_of(c * NPAD + lo, 8)
                pltpu.sync_copy(acc, out.at[pl.ds(ooff, TB)])
                return 0

            lax.fori_loop(0, Cc, chunk_body, 0)

        pl.run_scoped(phase2,
                      pltpu.VMEM((TB, 128), f32),
                      pltpu.VMEM((FB,), f32),
                      pltpu.VMEM((BE,), i32),
                      pltpu.VMEM((BE, 128), f32))

    return pl.kernel(
        body,
        mesh=_mesh,
        out_type=[
            jax.ShapeDtypeStruct((Cc * NPAD, 128), f32),
            jax.ShapeDtypeStruct((NT * H * CAPE,), f32),
        ],
        scratch_types=[
            pltpu.VMEM((TB * H,), f32),
            pltpu.VMEM((FB,), i32),
            pltpu.VMEM((FB,), i32),
            pltpu.VMEM((H * FB,), f32),
            pltpu.VMEM((NT * 16,), i32),
            pltpu.SemaphoreType.DMA,
            pltpu.SemaphoreType.DMA,
        ],
        compiler_params=_sc_params,
    )


def _make_mm(Cin_c, Cout_c, H, act, BN=1024):
    grid = (NPAD // BN, Cout_c, Cin_c)
    cph = Cout_c // max(H, 1)

    def body(*refs):
        if act == "none":
            (xin, w), rest = refs[:2], refs[2:]
        else:
            (xin, brow, w), rest = refs[:3], refs[3:]
        if H > 0:
            arows_s, arows_d, hc, asad = rest
        else:
            (hc,) = rest
        kc = pl.program_id(2)
        c = pl.program_id(1)
        a = xin[0]
        if act != "none":
            a = a + brow[0, 0][None, :]
        if act == "relu_bias":
            a = jnp.maximum(a, 0.0)
        part = jnp.dot(a, w[...], preferred_element_type=f32)

        @pl.when(kc == 0)
        def _():
            hc[0] = part

        @pl.when(kc > 0)
        def _():
            hc[0] = hc[0] + part

        if H > 0:
            @pl.when(kc == Cin_c - 1)
            def _():
                h = hc[0]
                head = c // cph
                lane = lax.broadcasted_iota(i32, (1, 8), 1)
                vs = jnp.sum(h * arows_s[0, 0][None, :], axis=1, keepdims=True)
                vd = jnp.sum(h * arows_d[0, 0][None, :], axis=1, keepdims=True)
                contrib = (jnp.where(lane == head, 1.0, 0.0) * vs
                           + jnp.where(lane == head + 4, 1.0, 0.0) * vd)

                @pl.when(c == 0)
                def _():
                    asad[...] = contrib

                @pl.when(c > 0)
                def _():
                    asad[...] = asad[...] + contrib

    in_specs = [pl.BlockSpec((1, BN, 128), lambda nb, c, kc: (kc, nb, 0))]
    if act != "none":
        in_specs.append(pl.BlockSpec((1, 1, 128), lambda nb, c, kc: (kc, 0, 0)))
    in_specs.append(pl.BlockSpec((128, 128), lambda nb, c, kc: (kc, c)))
    out_shapes = []
    out_specs = []
    if H > 0:
        in_specs.append(pl.BlockSpec((1, 1, 128), lambda nb, c, kc: (c, 0, 0)))
        in_specs.append(pl.BlockSpec((1, 1, 128), lambda nb, c, kc: (c, 0, 0)))
    out_shapes.append(jax.ShapeDtypeStruct((Cout_c, NPAD, 128), f32))
    out_specs.append(pl.BlockSpec((1, BN, 128), lambda nb, c, kc: (c, nb, 0)))
    if H > 0:
        out_shapes.append(jax.ShapeDtypeStruct((NPAD, 8), f32))
        out_specs.append(pl.BlockSpec((BN, 8), lambda nb, c, kc: (nb, 0)))

    return pl.pallas_call(
        body,
        grid=grid,
        in_specs=in_specs,
        out_specs=out_specs,
        out_shape=out_shapes,
    )


def kernel(x, edge_index, W1, a_src1, a_dst1, b1, W2, a_src2, a_dst2, b2,
           W3, a_src3, a_dst3, b3, Wg, bg, Wn, bn):
    xc = jnp.zeros((2, NPAD, 128), f32)
    xc = xc.at[:, :N, :].set(x.reshape(N, 2, 128).transpose(1, 0, 2))
    src_l, doff_l, cnt = _compact(edge_index[0], edge_index[1])

    def layer(xin, W, a_s, a_d, brow, Cin_c, Cout_c, H, act):
        arows_s = a_s.reshape(Cout_c, 1, 128)
        arows_d = a_d.reshape(Cout_c, 1, 128)
        mm = _make_mm(Cin_c, Cout_c, H, act)
        if act == "none":
            hc, asad = mm(xin, W, arows_s, arows_d)
        else:
            hc, asad = mm(xin, brow, W, arows_s, arows_d)
        hflat = hc.reshape(Cout_c * NPAD, 128)
        outc, _ = _make_gat_sc(Cout_c, H)(hflat, asad.reshape(NPAD * 8),
                                          src_l, doff_l, cnt)
        return outc.reshape(Cout_c, NPAD, 128)

    out1 = layer(xc, W1, a_src1, a_dst1, None, 2, 16, 4, "none")
    out2 = layer(out1, W2, a_src2, a_dst2, b1.reshape(16, 1, 128),
                 16, 8, 2, "relu_bias")
    out3 = layer(out2, W3, a_src3, a_dst3, b2.reshape(8, 1, 128),
                 8, 4, 1, "relu_bias")

    Wgn = jnp.zeros((512, 128), f32)
    Wgn = Wgn.at[:, :3].set(Wg).at[:, 3:4].set(Wn)
    bgn = jnp.zeros((128,), f32).at[:3].set(bg).at[3:4].set(bn)

    def fin_body(xin, brow, w, bglob, y):
        kc = pl.program_id(1)
        a = xin[0] + brow[0, 0][None, :]
        part = jnp.dot(a, w[...], preferred_element_type=f32)

        @pl.when(kc == 0)
        def _():
            y[...] = part + bglob[0][None, :]

        @pl.when(kc > 0)
        def _():
            y[...] = y[...] + part

    BN = 1024
    y = pl.pallas_call(
        fin_body,
        grid=(NPAD // BN, 4),
        in_specs=[
            pl.BlockSpec((1, BN, 128), lambda nb, kc: (kc, nb, 0)),
            pl.BlockSpec((1, 1, 128), lambda nb, kc: (kc, 0, 0)),
            pl.BlockSpec((128, 128), lambda nb, kc: (kc, 0)),
            pl.BlockSpec((1, 128), lambda nb, kc: (0, 0)),
        ],
        out_specs=pl.BlockSpec((BN, 128), lambda nb, kc: (nb, 0)),
        out_shape=jax.ShapeDtypeStruct((NPAD, 128), f32),
    )(out3, b3.reshape(4, 1, 128), Wgn, bgn.reshape(1, 128))

    return (y[:N, :3], y[:N, 3:4])


# speedup vs baseline: 2.4559x; 2.4559x over previous
import functools

import jax
import jax.numpy as jnp
from jax import lax
from jax.experimental import pallas as pl
from jax.experimental.pallas import tpu as pltpu
from jax.experimental.pallas import tpu_sc as plsc

N = 10000
E = 160000
NT = 32
TB = 320
NPAD = 10240
CE = 2000
FB = 2048
CAPE = FB * 80
BE = 128

_mesh = plsc.VectorSubcoreMesh(core_axis_name="c", subcore_axis_name="s")
_sc_params = pltpu.CompilerParams(needs_layout_passes=False)

f32 = jnp.float32
i32 = jnp.int32


def _wid():
    return lax.axis_index("s") * 2 + lax.axis_index("c")


def _iota16():
    return lax.iota(i32, 16)


def _compact_body(src_g, dst_g, src_o, doff_o, cnt_o, s_ch, d_ch, stg_s, stg_d, cntv):
    t = _wid()
    lo = t * TB
    hi = jnp.minimum(lo + TB, N)

    def append_vreg(cur, sv, dv, m):
        ranks = plsc.cumsum(m.astype(i32))
        idx = cur + ranks - 1
        plsc.store_scatter(stg_s, [idx], sv, mask=m)
        plsc.store_scatter(stg_d, [idx], dv - lo, mask=m)
        return cur + ranks[15]

    def maybe_flush(carry):
        cur, nf = carry

        def do_flush(c):
            foff = pl.multiple_of(t * CAPE + nf * FB, 8)
            pltpu.sync_copy(stg_s.at[pl.ds(0, FB)],
                            src_o.at[pl.ds(foff, FB)])
            pltpu.sync_copy(stg_d.at[pl.ds(0, FB)],
                            doff_o.at[pl.ds(foff, FB)])
            nmv = c - FB

            def mv(j, _):
                v = stg_s[pl.ds(FB + j * 16, 16)]
                stg_s[pl.ds(j * 16, 16)] = v
                v2 = stg_d[pl.ds(FB + j * 16, 16)]
                stg_d[pl.ds(j * 16, 16)] = v2
                return 0

            lax.fori_loop(0, (nmv + 15) // 16, mv, 0)
            return (c - FB, nf + 1)

        return lax.cond(cur >= FB, do_flush, lambda c: (c, nf), cur)

    def chunk_body(k, carry):
        cur, nf = carry
        off = pl.multiple_of(k * CE, 8)
        pltpu.sync_copy(src_g.at[pl.ds(off, CE)], s_ch)
        pltpu.sync_copy(dst_g.at[pl.ds(off, CE)], d_ch)

        def vec_body(i, c):
            sv = s_ch[pl.ds(i * 16, 16)]
            dv = d_ch[pl.ds(i * 16, 16)]
            m = (dv >= lo) & (dv < lo + TB)
            return append_vreg(c, sv, dv, m)

        cur = lax.fori_loop(0, CE // 16, vec_body, cur)
        return maybe_flush((cur, nf))

    cur, nf = lax.fori_loop(0, E // CE, chunk_body, (0, 0))

    def self_body(i, carry):
        c, f = carry
        dg = lo + i * 16 + _iota16()
        m = dg < hi
        c = append_vreg(c, dg, dg, m)
        return maybe_flush((c, f))

    cur, nf = lax.fori_loop(0, TB // 16, self_body, (cur, nf))
    total = nf * FB + cur
    foff = pl.multiple_of(t * CAPE + nf * FB, 8)
    pltpu.sync_copy(stg_s.at[pl.ds(0, FB)], src_o.at[pl.ds(foff, FB)])
    pltpu.sync_copy(stg_d.at[pl.ds(0, FB)], doff_o.at[pl.ds(foff, FB)])
    cntv[...] = jnp.broadcast_to(total, (16,)).astype(i32)
    pltpu.sync_copy(cntv, cnt_o.at[pl.ds(t * 16, 16)])


_compact = functools.partial(
    pl.kernel,
    mesh=_mesh,
    out_type=[
        jax.ShapeDtypeStruct((NT * CAPE,), i32),
        jax.ShapeDtypeStruct((NT * CAPE,), i32),
        jax.ShapeDtypeStruct((NT * 16,), i32),
    ],
    scratch_types=[
        pltpu.VMEM((CE,), i32),
        pltpu.VMEM((CE,), i32),
        pltpu.VMEM((2 * FB,), i32),
        pltpu.VMEM((2 * FB,), i32),
        pltpu.VMEM((16,), i32),
    ],
    compiler_params=_sc_params,
)(_compact_body)


def _make_gat_sc(Cc, H):
    def body(hc, asad, src_l, doff_l, cnt_i, out, w_l,
             rdenom, sstg, dstg, wstg, cntv, sem, sem2):
        t = _wid()
        lo = t * TB
        ebase = t * CAPE
        pltpu.sync_copy(cnt_i, cntv)
        cnt = cntv[pl.ds(t * 16, 16)][0]
        nblk = (cnt + FB - 1) // FB

        def phase1(asad_t, denom):
            pltpu.sync_copy(asad, asad_t)

            def z(i, _):
                denom[pl.ds(i * 16, 16)] = jnp.zeros((16,), f32)
                return 0

            lax.fori_loop(0, TB * H, z, 0)

            def blk_body(blk, _):
                boff = pl.multiple_of(ebase + blk * FB, 8)
                pltpu.sync_copy(src_l.at[pl.ds(boff, FB)], sstg)
                pltpu.sync_copy(doff_l.at[pl.ds(boff, FB)], dstg)

                def vec_body(i, _):
                    e0 = blk * FB + i * 16
                    m = (e0 + _iota16()) < cnt
                    sv = jnp.where(m, sstg[pl.ds(i * 16, 16)], 0)
                    dv = jnp.where(m, dstg[pl.ds(i * 16, 16)], 0)
                    dg = dv + lo
                    for h in range(H):
                        a1 = plsc.load_gather(asad_t, [sv * 8 + h])
                        a2 = plsc.load_gather(asad_t, [dg * 8 + 4 + h])
                        al = a1 + a2
                        al = jnp.maximum(al, 0.2 * al)
                        w = jnp.exp(al)
                        wstg[pl.ds(h * FB + i * 16, 16)] = w
                        plsc.addupdate_scatter(
                            denom, [_iota16() * (TB * H) + dv * H + h], w, mask=m)
                    return 0

                lax.fori_loop(0, FB // 16, vec_body, 0)
                for h in range(H):
                    woff = pl.multiple_of(t * (H * CAPE) + h * CAPE + blk * FB, 8)
                    pltpu.sync_copy(wstg.at[pl.ds(h * FB, FB)], w_l.at[pl.ds(woff, FB)])
                return 0

            lax.fori_loop(0, nblk, blk_body, 0)

            def rd(j, _):
                s = jnp.zeros((16,), f32)
                for st in range(16):
                    s = s + denom[pl.ds(st * (TB * H) + j * 16, 16)]
                rdenom[pl.ds(j * 16, 16)] = 1.0 / (s + 1e-16)
                return 0

            lax.fori_loop(0, (TB * H) // 16, rd, 0)

        pl.run_scoped(phase1,
                      pltpu.VMEM((NPAD * 8,), f32),
                      pltpu.VMEM((16 * TB * H,), f32))

        def phase2(acc, wb2, idxb, rows):
            def chunk_body(c, _):
                head = c >> 2

                def zacc(j, _):
                    acc[j >> 3, pl.ds((j & 7) * 16, 16)] = jnp.zeros((16,), f32)
                    return 0

                lax.fori_loop(0, TB * 8, zacc, 0)

                def batch_body(b, _):
                    boff = pl.multiple_of(ebase + b * BE, 8)
                    pltpu.sync_copy(src_l.at[pl.ds(boff, BE)], sstg.at[pl.ds(0, BE)])
                    pltpu.sync_copy(doff_l.at[pl.ds(boff, BE)], dstg.at[pl.ds(0, BE)])
                    woff = pl.multiple_of(t * (H * CAPE) + head * CAPE + b * BE, 8)
                    pltpu.sync_copy(w_l.at[pl.ds(woff, BE)], wb2)

                    def mkidx(i, _):
                        m = (b * BE + i * 16 + _iota16()) < cnt
                        sv = jnp.where(m, sstg[pl.ds(i * 16, 16)], 0)
                        idxb[pl.ds(i * 16, 16)] = sv + c * NPAD
                        return 0

                    lax.fori_loop(0, BE // 16, mkidx, 0)
                    pltpu.async_copy(hc.at[idxb], rows, sem).wait()

                    def edge_vec_body(i, _):
                        base = i * 16
                        m = (b * BE + base + _iota16()) < cnt
                        do16 = jnp.where(m, dstg[pl.ds(base, 16)], 0)
                        w16 = wb2[pl.ds(base, 16)]
                        r16 = plsc.load_gather(rdenom, [do16 * H + head])
                        att16 = jnp.where(m, w16 * r16, 0.0)
                        for jj in range(16):
                            att = att16[jj]
                            do = do16[jj]
                            for k in range(8):
                                v = rows[base + jj, pl.ds(k * 16, 16)] * att
                                plsc.addupdate(acc.at[do, pl.ds(k * 16, 16)], v)
                        return 0

                    lax.fori_loop(0, BE // 16, edge_vec_body, 0)
                    return 0

                lax.fori_loop(0, (cnt + BE - 1) // BE, batch_body, 0)
                ooff = pl.multiple_of(c * NPAD + lo, 8)
                pltpu.sync_copy(acc, out.at[pl.ds(ooff, TB)])
                return 0

            lax.fori_loop(0, Cc, chunk_body, 0)

        pl.run_scoped(phase2,
                      pltpu.VMEM((TB, 128), f32),
                      pltpu.VMEM((BE,), f32),
                      pltpu.VMEM((BE,), i32),
                      pltpu.VMEM((BE, 128), f32))

    return pl.kernel(
        body,
        mesh=_mesh,
        out_type=[
            jax.ShapeDtypeStruct((Cc * NPAD, 128), f32),
            jax.ShapeDtypeStruct((NT * H * CAPE,), f32),
        ],
        scratch_types=[
            pltpu.VMEM((TB * H,), f32),
            pltpu.VMEM((FB,), i32),
            pltpu.VMEM((FB,), i32),
            pltpu.VMEM((H * FB,), f32),
            pltpu.VMEM((NT * 16,), i32),
            pltpu.SemaphoreType.DMA,
            pltpu.SemaphoreType.DMA,
        ],
        compiler_params=_sc_params,
    )


def _make_mm(Cin_c, Cout_c, H, act, BN=1024):
    grid = (NPAD // BN, Cout_c, Cin_c)
    cph = Cout_c // max(H, 1)

    def body(*refs):
        if act == "none":
            (xin, w), rest = refs[:2], refs[2:]
        else:
            (xin, brow, w), rest = refs[:3], refs[3:]
        if H > 0:
            arows_s, arows_d, hc, asad = rest
        else:
            (hc,) = rest
        kc = pl.program_id(2)
        c = pl.program_id(1)
        a = xin[0]
        if act != "none":
            a = a + brow[0, 0][None, :]
        if act == "relu_bias":
            a = jnp.maximum(a, 0.0)
        part = jnp.dot(a, w[...], preferred_element_type=f32)

        @pl.when(kc == 0)
        def _():
            hc[0] = part

        @pl.when(kc > 0)
        def _():
            hc[0] = hc[0] + part

        if H > 0:
            @pl.when(kc == Cin_c - 1)
            def _():
                h = hc[0]
                head = c // cph
                lane = lax.broadcasted_iota(i32, (1, 8), 1)
                vs = jnp.sum(h * arows_s[0, 0][None, :], axis=1, keepdims=True)
                vd = jnp.sum(h * arows_d[0, 0][None, :], axis=1, keepdims=True)
                contrib = (jnp.where(lane == head, 1.0, 0.0) * vs
                           + jnp.where(lane == head + 4, 1.0, 0.0) * vd)

                @pl.when(c == 0)
                def _():
                    asad[...] = contrib

                @pl.when(c > 0)
                def _():
                    asad[...] = asad[...] + contrib

    in_specs = [pl.BlockSpec((1, BN, 128), lambda nb, c, kc: (kc, nb, 0))]
    if act != "none":
        in_specs.append(pl.BlockSpec((1, 1, 128), lambda nb, c, kc: (kc, 0, 0)))
    in_specs.append(pl.BlockSpec((128, 128), lambda nb, c, kc: (kc, c)))
    out_shapes = []
    out_specs = []
    if H > 0:
        in_specs.append(pl.BlockSpec((1, 1, 128), lambda nb, c, kc: (c, 0, 0)))
        in_specs.append(pl.BlockSpec((1, 1, 128), lambda nb, c, kc: (c, 0, 0)))
    out_shapes.append(jax.ShapeDtypeStruct((Cout_c, NPAD, 128), f32))
    out_specs.append(pl.BlockSpec((1, BN, 128), lambda nb, c, kc: (c, nb, 0)))
    if H > 0:
        out_shapes.append(jax.ShapeDtypeStruct((NPAD, 8), f32))
        out_specs.append(pl.BlockSpec((BN, 8), lambda nb, c, kc: (nb, 0)))

    return pl.pallas_call(
        body,
        grid=grid,
        in_specs=in_specs,
        out_specs=out_specs,
        out_shape=out_shapes,
    )


def kernel(x, edge_index, W1, a_src1, a_dst1, b1, W2, a_src2, a_dst2, b2,
           W3, a_src3, a_dst3, b3, Wg, bg, Wn, bn):
    xc = jnp.zeros((2, NPAD, 128), f32)
    xc = xc.at[:, :N, :].set(x.reshape(N, 2, 128).transpose(1, 0, 2))
    src_l, doff_l, cnt = _compact(edge_index[0], edge_index[1])

    def layer(xin, W, a_s, a_d, brow, Cin_c, Cout_c, H, act):
        arows_s = a_s.reshape(Cout_c, 1, 128)
        arows_d = a_d.reshape(Cout_c, 1, 128)
        mm = _make_mm(Cin_c, Cout_c, H, act)
        if act == "none":
            hc, asad = mm(xin, W, arows_s, arows_d)
        else:
            hc, asad = mm(xin, brow, W, arows_s, arows_d)
        hflat = hc.reshape(Cout_c * NPAD, 128)
        outc, _ = _make_gat_sc(Cout_c, H)(hflat, asad.reshape(NPAD * 8),
                                          src_l, doff_l, cnt)
        return outc.reshape(Cout_c, NPAD, 128)

    out1 = layer(xc, W1, a_src1, a_dst1, None, 2, 16, 4, "none")
    out2 = layer(out1, W2, a_src2, a_dst2, b1.reshape(16, 1, 128),
                 16, 8, 2, "relu_bias")
    out3 = layer(out2, W3, a_src3, a_dst3, b2.reshape(8, 1, 128),
                 8, 4, 1, "relu_bias")

    Wgn = jnp.zeros((512, 128), f32)
    Wgn = Wgn.at[:, :3].set(Wg).at[:, 3:4].set(Wn)
    bgn = jnp.zeros((128,), f32).at[:3].set(bg).at[3:4].set(bn)

    def fin_body(xin, brow, w, bglob, y):
        kc = pl.program_id(1)
        a = xin[0] + brow[0, 0][None, :]
        part = jnp.dot(a, w[...], preferred_element_type=f32)

        @pl.when(kc == 0)
        def _():
            y[...] = part + bglob[0][None, :]

        @pl.when(kc > 0)
        def _():
            y[...] = y[...] + part

    BN = 1024
    y = pl.pallas_call(
        fin_body,
        grid=(NPAD // BN, 4),
        in_specs=[
            pl.BlockSpec((1, BN, 128), lambda nb, kc: (kc, nb, 0)),
            pl.BlockSpec((1, 1, 128), lambda nb, kc: (kc, 0, 0)),
            pl.BlockSpec((128, 128), lambda nb, kc: (kc, 0)),
            pl.BlockSpec((1, 128), lambda nb, kc: (0, 0)),
        ],
        out_specs=pl.BlockSpec((BN, 128), lambda nb, kc: (nb, 0)),
        out_shape=jax.ShapeDtypeStruct((NPAD, 128), f32),
    )(out3, b3.reshape(4, 1, 128), Wgn, bgn.reshape(1, 128))

    return (y[:N, :3], y[:N, 3:4])


# speedup vs baseline: 2.4820x; 1.0106x over previous
import functools

import jax
import jax.numpy as jnp
from jax import lax
from jax.experimental import pallas as pl
from jax.experimental.pallas import tpu as pltpu
from jax.experimental.pallas import tpu_sc as plsc

N = 10000
E = 160000
NT = 32
TB = 320
NPAD = 10240
CE = 2000
FB = 2048
CAPE = FB * 80
BE = 128

_mesh = plsc.VectorSubcoreMesh(core_axis_name="c", subcore_axis_name="s")
_sc_params = pltpu.CompilerParams(needs_layout_passes=False)

f32 = jnp.float32
i32 = jnp.int32


def _wid():
    return lax.axis_index("s") * 2 + lax.axis_index("c")


def _iota16():
    return lax.iota(i32, 16)


def _compact_body(src_g, dst_g, src_o, doff_o, cnt_o, s_ch, d_ch, stg_s, stg_d, cntv):
    t = _wid()
    lo = t * TB
    hi = jnp.minimum(lo + TB, N)

    def append_vreg(cur, sv, dv, m):
        ranks = plsc.cumsum(m.astype(i32))
        idx = cur + ranks - 1
        plsc.store_scatter(stg_s, [idx], sv, mask=m)
        plsc.store_scatter(stg_d, [idx], dv - lo, mask=m)
        return cur + ranks[15]

    def maybe_flush(carry):
        cur, nf = carry

        def do_flush(c):
            foff = pl.multiple_of(t * CAPE + nf * FB, 8)
            pltpu.sync_copy(stg_s.at[pl.ds(0, FB)],
                            src_o.at[pl.ds(foff, FB)])
            pltpu.sync_copy(stg_d.at[pl.ds(0, FB)],
                            doff_o.at[pl.ds(foff, FB)])
            nmv = c - FB

            def mv(j, _):
                v = stg_s[pl.ds(FB + j * 16, 16)]
                stg_s[pl.ds(j * 16, 16)] = v
                v2 = stg_d[pl.ds(FB + j * 16, 16)]
                stg_d[pl.ds(j * 16, 16)] = v2
                return 0

            lax.fori_loop(0, (nmv + 15) // 16, mv, 0)
            return (c - FB, nf + 1)

        return lax.cond(cur >= FB, do_flush, lambda c: (c, nf), cur)

    def chunk_body(k, carry):
        cur, nf = carry
        off = pl.multiple_of(k * CE, 8)
        pltpu.sync_copy(src_g.at[pl.ds(off, CE)], s_ch)
        pltpu.sync_copy(dst_g.at[pl.ds(off, CE)], d_ch)

        def vec_body(i, c):
            sv = s_ch[pl.ds(i * 16, 16)]
            dv = d_ch[pl.ds(i * 16, 16)]
            m = (dv >= lo) & (dv < lo + TB)
            return append_vreg(c, sv, dv, m)

        cur = lax.fori_loop(0, CE // 16, vec_body, cur)
        return maybe_flush((cur, nf))

    cur, nf = lax.fori_loop(0, E // CE, chunk_body, (0, 0))

    def self_body(i, carry):
        c, f = carry
        dg = lo + i * 16 + _iota16()
        m = dg < hi
        c = append_vreg(c, dg, dg, m)
        return maybe_flush((c, f))

    cur, nf = lax.fori_loop(0, TB // 16, self_body, (cur, nf))
    total = nf * FB + cur
    foff = pl.multiple_of(t * CAPE + nf * FB, 8)
    pltpu.sync_copy(stg_s.at[pl.ds(0, FB)], src_o.at[pl.ds(foff, FB)])
    pltpu.sync_copy(stg_d.at[pl.ds(0, FB)], doff_o.at[pl.ds(foff, FB)])
    cntv[...] = jnp.broadcast_to(total, (16,)).astype(i32)
    pltpu.sync_copy(cntv, cnt_o.at[pl.ds(t * 16, 16)])


_compact = functools.partial(
    pl.kernel,
    mesh=_mesh,
    out_type=[
        jax.ShapeDtypeStruct((NT * CAPE,), i32),
        jax.ShapeDtypeStruct((NT * CAPE,), i32),
        jax.ShapeDtypeStruct((NT * 16,), i32),
    ],
    scratch_types=[
        pltpu.VMEM((CE,), i32),
        pltpu.VMEM((CE,), i32),
        pltpu.VMEM((2 * FB,), i32),
        pltpu.VMEM((2 * FB,), i32),
        pltpu.VMEM((16,), i32),
    ],
    compiler_params=_sc_params,
)(_compact_body)


def _make_gat_sc(Cc, H):
    def body(hc, asad, src_l, doff_l, cnt_i, out, w_l,
             rdenom, sstg, dstg, wstg, cntv, sem, sem2):
        t = _wid()
        lo = t * TB
        ebase = t * CAPE
        pltpu.sync_copy(cnt_i, cntv)
        cnt = cntv[pl.ds(t * 16, 16)][0]
        nblk = (cnt + FB - 1) // FB

        def phase1(asad_t, denom):
            pltpu.sync_copy(asad, asad_t)

            def z(i, _):
                denom[pl.ds(i * 16, 16)] = jnp.zeros((16,), f32)
                return 0

            lax.fori_loop(0, TB * H, z, 0)

            def blk_body(blk, _):
                boff = pl.multiple_of(ebase + blk * FB, 8)
                pltpu.sync_copy(src_l.at[pl.ds(boff, FB)], sstg)
                pltpu.sync_copy(doff_l.at[pl.ds(boff, FB)], dstg)

                def vec_body(i, _):
                    e0 = blk * FB + i * 16
                    m = (e0 + _iota16()) < cnt
                    sv = jnp.where(m, sstg[pl.ds(i * 16, 16)], 0)
                    dv = jnp.where(m, dstg[pl.ds(i * 16, 16)], 0)
                    dg = dv + lo
                    for h in range(H):
                        a1 = plsc.load_gather(asad_t, [sv * 8 + h])
                        a2 = plsc.load_gather(asad_t, [dg * 8 + 4 + h])
                        al = a1 + a2
                        al = jnp.maximum(al, 0.2 * al)
                        w = jnp.exp(al)
                        wstg[pl.ds(h * FB + i * 16, 16)] = w
                        plsc.addupdate_scatter(
                            denom, [_iota16() * (TB * H) + dv * H + h], w, mask=m)
                    return 0

                lax.fori_loop(0, FB // 16, vec_body, 0)
                for h in range(H):
                    woff = pl.multiple_of(t * (H * CAPE) + h * CAPE + blk * FB, 8)
                    pltpu.sync_copy(wstg.at[pl.ds(h * FB, FB)], w_l.at[pl.ds(woff, FB)])
                return 0

            lax.fori_loop(0, nblk, blk_body, 0)

            def rd(j, _):
                s = jnp.zeros((16,), f32)
                for st in range(16):
                    s = s + denom[pl.ds(st * (TB * H) + j * 16, 16)]
                rdenom[pl.ds(j * 16, 16)] = 1.0 / (s + 1e-16)
                return 0

            lax.fori_loop(0, (TB * H) // 16, rd, 0)

        pl.run_scoped(phase1,
                      pltpu.VMEM((NPAD * 8,), f32),
                      pltpu.VMEM((16 * TB * H,), f32))

        def phase2(acc, wb2, idxb, rows, wb2b, idxb2, rows2):
            def chunk_body(c, _):
                head = c >> 2

                def zacc(j, _):
                    acc[j >> 3, pl.ds((j & 7) * 16, 16)] = jnp.zeros((16,), f32)
                    return 0

                lax.fori_loop(0, TB * 8, zacc, 0)

                def fetch(b, soff, wbuf, ibuf, rbuf, s):
                    boff = pl.multiple_of(ebase + b * BE, 8)
                    pltpu.sync_copy(src_l.at[pl.ds(boff, BE)],
                                    sstg.at[pl.ds(soff, BE)])
                    pltpu.sync_copy(doff_l.at[pl.ds(boff, BE)],
                                    dstg.at[pl.ds(soff, BE)])
                    woff = pl.multiple_of(t * (H * CAPE) + head * CAPE + b * BE, 8)
                    pltpu.sync_copy(w_l.at[pl.ds(woff, BE)], wbuf)

                    def mkidx(i, _):
                        m = (b * BE + i * 16 + _iota16()) < cnt
                        sv = jnp.where(m, sstg[pl.ds(soff + i * 16, 16)], 0)
                        ibuf[pl.ds(i * 16, 16)] = sv + c * NPAD
                        return 0

                    lax.fori_loop(0, BE // 16, mkidx, 0)
                    pltpu.async_copy(hc.at[ibuf], rbuf, s)

                def process(b, soff, wbuf, ibuf, rbuf, s):
                    pltpu.make_async_copy(hc.at[ibuf], rbuf, s).wait()

                    def edge_vec_body(i, _):
                        base = i * 16
                        m = (b * BE + base + _iota16()) < cnt
                        do16 = jnp.where(m, dstg[pl.ds(soff + base, 16)], 0)
                        w16 = wbuf[pl.ds(base, 16)]
                        r16 = plsc.load_gather(rdenom, [do16 * H + head])
                        att16 = jnp.where(m, w16 * r16, 0.0)
                        for jj in range(16):
                            att = att16[jj]
                            do = do16[jj]
                            for k in range(8):
                                v = rbuf[base + jj, pl.ds(k * 16, 16)] * att
                                plsc.addupdate(acc.at[do, pl.ds(k * 16, 16)], v)
                        return 0

                    lax.fori_loop(0, BE // 16, edge_vec_body, 0)

                def pair_body(p, _):
                    b0 = 2 * p
                    fetch(b0, 0, wb2, idxb, rows, sem)
                    fetch(b0 + 1, BE, wb2b, idxb2, rows2, sem2)
                    process(b0, 0, wb2, idxb, rows, sem)
                    process(b0 + 1, BE, wb2b, idxb2, rows2, sem2)
                    return 0

                npair = (cnt + 2 * BE - 1) // (2 * BE)
                lax.fori_loop(0, npair, pair_body, 0)
                ooff = pl.multiple_of(c * NPAD + lo, 8)
                pltpu.sync_copy(acc, out.at[pl.ds(ooff, TB)])
                return 0

            lax.fori_loop(0, Cc, chunk_body, 0)

        pl.run_scoped(phase2,
                      pltpu.VMEM((TB, 128), f32),
                      pltpu.VMEM((BE,), f32),
                      pltpu.VMEM((BE,), i32),
                      pltpu.VMEM((BE, 128), f32),
                      pltpu.VMEM((BE,), f32),
                      pltpu.VMEM((BE,), i32),
                      pltpu.VMEM((BE, 128), f32))

    return pl.kernel(
        body,
        mesh=_mesh,
        out_type=[
            jax.ShapeDtypeStruct((Cc * NPAD, 128), f32),
            jax.ShapeDtypeStruct((NT * H * CAPE,), f32),
        ],
        scratch_types=[
            pltpu.VMEM((TB * H,), f32),
            pltpu.VMEM((FB,), i32),
            pltpu.VMEM((FB,), i32),
            pltpu.VMEM((H * FB,), f32),
            pltpu.VMEM((NT * 16,), i32),
            pltpu.SemaphoreType.DMA,
            pltpu.SemaphoreType.DMA,
        ],
        compiler_params=_sc_params,
    )


def _make_mm(Cin_c, Cout_c, H, act, BN=1024):
    grid = (NPAD // BN, Cout_c, Cin_c)
    cph = Cout_c // max(H, 1)

    def body(*refs):
        if act == "none":
            (xin, w), rest = refs[:2], refs[2:]
        else:
            (xin, brow, w), rest = refs[:3], refs[3:]
        if H > 0:
            arows_s, arows_d, hc, asad = rest
        else:
            (hc,) = rest
        kc = pl.program_id(2)
        c = pl.program_id(1)
        a = xin[0]
        if act != "none":
            a = a + brow[0, 0][None, :]
        if act == "relu_bias":
            a = jnp.maximum(a, 0.0)
        part = jnp.dot(a, w[...], preferred_element_type=f32)

        @pl.when(kc == 0)
        def _():
            hc[0] = part

        @pl.when(kc > 0)
        def _():
            hc[0] = hc[0] + part

        if H > 0:
            @pl.when(kc == Cin_c - 1)
            def _():
                h = hc[0]
                head = c // cph
                lane = lax.broadcasted_iota(i32, (1, 8), 1)
                vs = jnp.sum(h * arows_s[0, 0][None, :], axis=1, keepdims=True)
                vd = jnp.sum(h * arows_d[0, 0][None, :], axis=1, keepdims=True)
                contrib = (jnp.where(lane == head, 1.0, 0.0) * vs
                           + jnp.where(lane == head + 4, 1.0, 0.0) * vd)

                @pl.when(c == 0)
                def _():
                    asad[...] = contrib

                @pl.when(c > 0)
                def _():
                    asad[...] = asad[...] + contrib

    in_specs = [pl.BlockSpec((1, BN, 128), lambda nb, c, kc: (kc, nb, 0))]
    if act != "none":
        in_specs.append(pl.BlockSpec((1, 1, 128), lambda nb, c, kc: (kc, 0, 0)))
    in_specs.append(pl.BlockSpec((128, 128), lambda nb, c, kc: (kc, c)))
    out_shapes = []
    out_specs = []
    if H > 0:
        in_specs.append(pl.BlockSpec((1, 1, 128), lambda nb, c, kc: (c, 0, 0)))
        in_specs.append(pl.BlockSpec((1, 1, 128), lambda nb, c, kc: (c, 0, 0)))
    out_shapes.append(jax.ShapeDtypeStruct((Cout_c, NPAD, 128), f32))
    out_specs.append(pl.BlockSpec((1, BN, 128), lambda nb, c, kc: (c, nb, 0)))
    if H > 0:
        out_shapes.append(jax.ShapeDtypeStruct((NPAD, 8), f32))
        out_specs.append(pl.BlockSpec((BN, 8), lambda nb, c, kc: (nb, 0)))

    return pl.pallas_call(
        body,
        grid=grid,
        in_specs=in_specs,
        out_specs=out_specs,
        out_shape=out_shapes,
    )


def kernel(x, edge_index, W1, a_src1, a_dst1, b1, W2, a_src2, a_dst2, b2,
           W3, a_src3, a_dst3, b3, Wg, bg, Wn, bn):
    xc = jnp.zeros((2, NPAD, 128), f32)
    xc = xc.at[:, :N, :].set(x.reshape(N, 2, 128).transpose(1, 0, 2))
    src_l, doff_l, cnt = _compact(edge_index[0], edge_index[1])

    def layer(xin, W, a_s, a_d, brow, Cin_c, Cout_c, H, act):
        arows_s = a_s.reshape(Cout_c, 1, 128)
        arows_d = a_d.reshape(Cout_c, 1, 128)
        mm = _make_mm(Cin_c, Cout_c, H, act)
        if act == "none":
            hc, asad = mm(xin, W, arows_s, arows_d)
        else:
            hc, asad = mm(xin, brow, W, arows_s, arows_d)
        hflat = hc.reshape(Cout_c * NPAD, 128)
        outc, _ = _make_gat_sc(Cout_c, H)(hflat, asad.reshape(NPAD * 8),
                                          src_l, doff_l, cnt)
        return outc.reshape(Cout_c, NPAD, 128)

    out1 = layer(xc, W1, a_src1, a_dst1, None, 2, 16, 4, "none")
    out2 = layer(out1, W2, a_src2, a_dst2, b1.reshape(16, 1, 128),
                 16, 8, 2, "relu_bias")
    out3 = layer(out2, W3, a_src3, a_dst3, b2.reshape(8, 1, 128),
                 8, 4, 1, "relu_bias")

    Wgn = jnp.zeros((512, 128), f32)
    Wgn = Wgn.at[:, :3].set(Wg).at[:, 3:4].set(Wn)
    bgn = jnp.zeros((128,), f32).at[:3].set(bg).at[3:4].set(bn)

    def fin_body(xin, brow, w, bglob, y):
        kc = pl.program_id(1)
        a = xin[0] + brow[0, 0][None, :]
        part = jnp.dot(a, w[...], preferred_element_type=f32)

        @pl.when(kc == 0)
        def _():
            y[...] = part + bglob[0][None, :]

        @pl.when(kc > 0)
        def _():
            y[...] = y[...] + part

    BN = 1024
    y = pl.pallas_call(
        fin_body,
        grid=(NPAD // BN, 4),
        in_specs=[
            pl.BlockSpec((1, BN, 128), lambda nb, kc: (kc, nb, 0)),
            pl.BlockSpec((1, 1, 128), lambda nb, kc: (kc, 0, 0)),
            pl.BlockSpec((128, 128), lambda nb, kc: (kc, 0)),
            pl.BlockSpec((1, 128), lambda nb, kc: (0, 0)),
        ],
        out_specs=pl.BlockSpec((BN, 128), lambda nb, kc: (nb, 0)),
        out_shape=jax.ShapeDtypeStruct((NPAD, 128), f32),
    )(out3, b3.reshape(4, 1, 128), Wgn, bgn.reshape(1, 128))

    return (y[:N, :3], y[:N, 3:4])


# speedup vs baseline: 2.9237x; 1.1780x over previous
import functools

import jax
import jax.numpy as jnp
from jax import lax
from jax.experimental import pallas as pl
from jax.experimental.pallas import tpu as pltpu
from jax.experimental.pallas import tpu_sc as plsc

N = 10000
E = 160000
NT = 32
TB = 320
NPAD = 10240
CE = 2000
FB = 2048
CAPE = FB * 80
BE = 128

_mesh = plsc.VectorSubcoreMesh(core_axis_name="c", subcore_axis_name="s")
_sc_params = pltpu.CompilerParams(needs_layout_passes=False)

f32 = jnp.float32
i32 = jnp.int32


def _wid():
    return lax.axis_index("s") * 2 + lax.axis_index("c")


def _iota16():
    return lax.iota(i32, 16)


def _compact_body(src_g, dst_g, src_o, doff_o, cnt_o, s_ch, d_ch, stg_s, stg_d, cntv):
    t = _wid()
    lo = t * TB
    hi = jnp.minimum(lo + TB, N)

    def append_vreg(cur, sv, dv, m):
        ranks = plsc.cumsum(m.astype(i32))
        idx = cur + ranks - 1
        plsc.store_scatter(stg_s, [idx], sv, mask=m)
        plsc.store_scatter(stg_d, [idx], dv - lo, mask=m)
        return cur + ranks[15]

    def maybe_flush(carry):
        cur, nf = carry

        def do_flush(c):
            foff = pl.multiple_of(t * CAPE + nf * FB, 8)
            pltpu.sync_copy(stg_s.at[pl.ds(0, FB)],
                            src_o.at[pl.ds(foff, FB)])
            pltpu.sync_copy(stg_d.at[pl.ds(0, FB)],
                            doff_o.at[pl.ds(foff, FB)])
            nmv = c - FB

            def mv(j, _):
                v = stg_s[pl.ds(FB + j * 16, 16)]
                stg_s[pl.ds(j * 16, 16)] = v
                v2 = stg_d[pl.ds(FB + j * 16, 16)]
                stg_d[pl.ds(j * 16, 16)] = v2
                return 0

            lax.fori_loop(0, (nmv + 15) // 16, mv, 0)
            return (c - FB, nf + 1)

        return lax.cond(cur >= FB, do_flush, lambda c: (c, nf), cur)

    def chunk_body(k, carry):
        cur, nf = carry
        off = pl.multiple_of(k * CE, 8)
        pltpu.sync_copy(src_g.at[pl.ds(off, CE)], s_ch)
        pltpu.sync_copy(dst_g.at[pl.ds(off, CE)], d_ch)

        def vec_body(i, c):
            sv = s_ch[pl.ds(i * 16, 16)]
            dv = d_ch[pl.ds(i * 16, 16)]
            m = (dv >= lo) & (dv < lo + TB)
            return append_vreg(c, sv, dv, m)

        cur = lax.fori_loop(0, CE // 16, vec_body, cur)
        return maybe_flush((cur, nf))

    cur, nf = lax.fori_loop(0, E // CE, chunk_body, (0, 0))

    def self_body(i, carry):
        c, f = carry
        dg = lo + i * 16 + _iota16()
        m = dg < hi
        c = append_vreg(c, dg, dg, m)
        return maybe_flush((c, f))

    cur, nf = lax.fori_loop(0, TB // 16, self_body, (cur, nf))
    total = nf * FB + cur
    foff = pl.multiple_of(t * CAPE + nf * FB, 8)
    pltpu.sync_copy(stg_s.at[pl.ds(0, FB)], src_o.at[pl.ds(foff, FB)])
    pltpu.sync_copy(stg_d.at[pl.ds(0, FB)], doff_o.at[pl.ds(foff, FB)])
    cntv[...] = jnp.broadcast_to(total, (16,)).astype(i32)
    pltpu.sync_copy(cntv, cnt_o.at[pl.ds(t * 16, 16)])


_compact = functools.partial(
    pl.kernel,
    mesh=_mesh,
    out_type=[
        jax.ShapeDtypeStruct((NT * CAPE,), i32),
        jax.ShapeDtypeStruct((NT * CAPE,), i32),
        jax.ShapeDtypeStruct((NT * 16,), i32),
    ],
    scratch_types=[
        pltpu.VMEM((CE,), i32),
        pltpu.VMEM((CE,), i32),
        pltpu.VMEM((2 * FB,), i32),
        pltpu.VMEM((2 * FB,), i32),
        pltpu.VMEM((16,), i32),
    ],
    compiler_params=_sc_params,
)(_compact_body)


def _make_gat_sc(Cc, H):
    def body(hc, asad, src_l, doff_l, cnt_i, out, w_l,
             rdenom, sstg, dstg, wstg, cntv, sem, sem2):
        t = _wid()
        lo = t * TB
        ebase = t * CAPE
        pltpu.sync_copy(cnt_i, cntv)
        cnt = cntv[pl.ds(t * 16, 16)][0]
        nblk = (cnt + FB - 1) // FB

        def phase1(asad_t, denom):
            pltpu.sync_copy(asad, asad_t)

            def z(i, _):
                denom[pl.ds(i * 16, 16)] = jnp.zeros((16,), f32)
                return 0

            lax.fori_loop(0, TB * H, z, 0)

            def blk_body(blk, _):
                boff = pl.multiple_of(ebase + blk * FB, 8)
                pltpu.sync_copy(src_l.at[pl.ds(boff, FB)], sstg)
                pltpu.sync_copy(doff_l.at[pl.ds(boff, FB)], dstg)

                def vec_body(i, _):
                    e0 = blk * FB + i * 16
                    m = (e0 + _iota16()) < cnt
                    sv = jnp.where(m, sstg[pl.ds(i * 16, 16)], 0)
                    dv = jnp.where(m, dstg[pl.ds(i * 16, 16)], 0)
                    dg = dv + lo
                    for h in range(H):
                        a1 = plsc.load_gather(asad_t, [sv * 8 + h])
                        a2 = plsc.load_gather(asad_t, [dg * 8 + 4 + h])
                        al = a1 + a2
                        al = jnp.maximum(al, 0.2 * al)
                        w = jnp.exp(al)
                        wstg[pl.ds(h * FB + i * 16, 16)] = w
                        plsc.addupdate_scatter(
                            denom, [_iota16() * (TB * H) + dv * H + h], w, mask=m)
                    return 0

                lax.fori_loop(0, FB // 16, vec_body, 0)
                for h in range(H):
                    woff = pl.multiple_of(t * (H * CAPE) + h * CAPE + blk * FB, 8)
                    pltpu.sync_copy(wstg.at[pl.ds(h * FB, FB)], w_l.at[pl.ds(woff, FB)])
                return 0

            lax.fori_loop(0, nblk, blk_body, 0)

            def rd(j, _):
                s = jnp.zeros((16,), f32)
                for st in range(16):
                    s = s + denom[pl.ds(st * (TB * H) + j * 16, 16)]
                rdenom[pl.ds(j * 16, 16)] = 1.0 / (s + 1e-16)
                return 0

            lax.fori_loop(0, (TB * H) // 16, rd, 0)

        pl.run_scoped(phase1,
                      pltpu.VMEM((NPAD * 8,), f32),
                      pltpu.VMEM((16 * TB * H,), f32))

        def phase2(acc, wb2, idxb, rows, wb2b, idxb2, rows2):
            def chunk_body(c, _):
                head = c >> 2

                def zacc(j, _):
                    acc[j >> 3, pl.ds((j & 7) * 16, 16)] = jnp.zeros((16,), f32)
                    return 0

                lax.fori_loop(0, TB * 8, zacc, 0)

                def fetch(b, soff, wbuf, ibuf, rbuf, s):
                    boff = pl.multiple_of(ebase + b * BE, 8)
                    pltpu.sync_copy(src_l.at[pl.ds(boff, BE)],
                                    sstg.at[pl.ds(soff, BE)])
                    pltpu.sync_copy(doff_l.at[pl.ds(boff, BE)],
                                    dstg.at[pl.ds(soff, BE)])
                    woff = pl.multiple_of(t * (H * CAPE) + head * CAPE + b * BE, 8)
                    pltpu.sync_copy(w_l.at[pl.ds(woff, BE)], wbuf)

                    def mkidx(i, _):
                        m = (b * BE + i * 16 + _iota16()) < cnt
                        sv = jnp.where(m, sstg[pl.ds(soff + i * 16, 16)], 0)
                        ibuf[pl.ds(i * 16, 16)] = sv + c * NPAD
                        return 0

                    lax.fori_loop(0, BE // 16, mkidx, 0)
                    pltpu.async_copy(hc.at[ibuf], rbuf, s)

                def process(b, soff, wbuf, ibuf, rbuf, s):
                    pltpu.make_async_copy(hc.at[ibuf], rbuf, s).wait()

                    def edge_vec_body(i, _):
                        base = i * 16
                        m = (b * BE + base + _iota16()) < cnt
                        do16 = jnp.where(m, dstg[pl.ds(soff + base, 16)], 0)
                        w16 = wbuf[pl.ds(base, 16)]
                        r16 = plsc.load_gather(rdenom, [do16 * H + head])
                        att16 = jnp.where(m, w16 * r16, 0.0)
                        for jj in range(16):
                            att = att16[jj]
                            do = do16[jj]
                            for k in range(8):
                                v = rbuf[base + jj, pl.ds(k * 16, 16)] * att
                                plsc.addupdate(acc.at[do, pl.ds(k * 16, 16)], v)
                        return 0

                    lax.fori_loop(0, BE // 16, edge_vec_body, 0)

                def pair_body(p, _):
                    b0 = 2 * p
                    fetch(b0, 0, wb2, idxb, rows, sem)
                    fetch(b0 + 1, BE, wb2b, idxb2, rows2, sem2)
                    process(b0, 0, wb2, idxb, rows, sem)
                    process(b0 + 1, BE, wb2b, idxb2, rows2, sem2)
                    return 0

                npair = (cnt + 2 * BE - 1) // (2 * BE)
                lax.fori_loop(0, npair, pair_body, 0)
                ooff = pl.multiple_of(c * NPAD + lo, 8)
                pltpu.sync_copy(acc, out.at[pl.ds(ooff, TB)])
                return 0

            lax.fori_loop(0, Cc, chunk_body, 0)

        pl.run_scoped(phase2,
                      pltpu.VMEM((TB, 128), f32),
                      pltpu.VMEM((BE,), f32),
                      pltpu.VMEM((BE,), i32),
                      pltpu.VMEM((BE, 128), f32),
                      pltpu.VMEM((BE,), f32),
                      pltpu.VMEM((BE,), i32),
                      pltpu.VMEM((BE, 128), f32))

    return pl.kernel(
        body,
        mesh=_mesh,
        out_type=[
            jax.ShapeDtypeStruct((Cc * NPAD, 128), f32),
            jax.ShapeDtypeStruct((NT * H * CAPE,), f32),
        ],
        scratch_types=[
            pltpu.VMEM((TB * H,), f32),
            pltpu.VMEM((FB,), i32),
            pltpu.VMEM((FB,), i32),
            pltpu.VMEM((H * FB,), f32),
            pltpu.VMEM((NT * 16,), i32),
            pltpu.SemaphoreType.DMA,
            pltpu.SemaphoreType.DMA,
        ],
        compiler_params=_sc_params,
    )


def _make_mm(Cin_c, Cout_c, H, act, BN=1024):
    grid = (NPAD // BN, Cout_c)
    cph = Cout_c // max(H, 1)

    def body(*refs):
        if act == "none":
            (xin, w), rest = refs[:2], refs[2:]
        else:
            (xin, brow, w), rest = refs[:3], refs[3:]
        arows_s, arows_d, hc, asad = rest
        c = pl.program_id(1)
        accum = None
        for k in range(Cin_c):
            a = xin[k]
            if act != "none":
                a = a + brow[k, 0][None, :]
            if act == "relu_bias":
                a = jnp.maximum(a, 0.0)
            p = jnp.dot(a, w[0, k], preferred_element_type=f32)
            accum = p if accum is None else accum + p
        hc[0] = accum
        if H > 0:
            head = c // cph
            lane = lax.broadcasted_iota(i32, (1, 8), 1)
            vs = jnp.sum(accum * arows_s[0, 0][None, :], axis=1, keepdims=True)
            vd = jnp.sum(accum * arows_d[0, 0][None, :], axis=1, keepdims=True)
            contrib = (jnp.where(lane == head, 1.0, 0.0) * vs
                       + jnp.where(lane == head + 4, 1.0, 0.0) * vd)

            @pl.when(c == 0)
            def _():
                asad[...] = contrib

            @pl.when(c > 0)
            def _():
                asad[...] = asad[...] + contrib

    in_specs = [pl.BlockSpec((Cin_c, BN, 128), lambda nb, c: (0, nb, 0))]
    if act != "none":
        in_specs.append(pl.BlockSpec((Cin_c, 1, 128), lambda nb, c: (0, 0, 0)))
    in_specs.append(pl.BlockSpec((1, Cin_c, 128, 128), lambda nb, c: (c, 0, 0, 0)))
    in_specs.append(pl.BlockSpec((1, 1, 128), lambda nb, c: (c, 0, 0)))
    in_specs.append(pl.BlockSpec((1, 1, 128), lambda nb, c: (c, 0, 0)))
    out_shapes = [jax.ShapeDtypeStruct((Cout_c, NPAD, 128), f32)]
    out_specs = [pl.BlockSpec((1, BN, 128), lambda nb, c: (c, nb, 0))]
    out_shapes.append(jax.ShapeDtypeStruct((NPAD, 8), f32))
    out_specs.append(pl.BlockSpec((BN, 8), lambda nb, c: (nb, 0)))

    return pl.pallas_call(
        body,
        grid=grid,
        in_specs=in_specs,
        out_specs=out_specs,
        out_shape=out_shapes,
    )


def kernel(x, edge_index, W1, a_src1, a_dst1, b1, W2, a_src2, a_dst2, b2,
           W3, a_src3, a_dst3, b3, Wg, bg, Wn, bn):
    xc = jnp.zeros((2, NPAD, 128), f32)
    xc = xc.at[:, :N, :].set(x.reshape(N, 2, 128).transpose(1, 0, 2))
    src_l, doff_l, cnt = _compact(edge_index[0], edge_index[1])

    def layer(xin, W, a_s, a_d, brow, Cin_c, Cout_c, H, act):
        arows_s = a_s.reshape(Cout_c, 1, 128)
        arows_d = a_d.reshape(Cout_c, 1, 128)
        W4 = W.reshape(Cin_c, 128, Cout_c, 128).transpose(2, 0, 1, 3)
        mm = _make_mm(Cin_c, Cout_c, H, act)
        if act == "none":
            hc, asad = mm(xin, W4, arows_s, arows_d)
        else:
            hc, asad = mm(xin, brow, W4, arows_s, arows_d)
        hflat = hc.reshape(Cout_c * NPAD, 128)
        outc, _ = _make_gat_sc(Cout_c, H)(hflat, asad.reshape(NPAD * 8),
                                          src_l, doff_l, cnt)
        return outc.reshape(Cout_c, NPAD, 128)

    out1 = layer(xc, W1, a_src1, a_dst1, None, 2, 16, 4, "none")
    out2 = layer(out1, W2, a_src2, a_dst2, b1.reshape(16, 1, 128),
                 16, 8, 2, "relu_bias")
    out3 = layer(out2, W3, a_src3, a_dst3, b2.reshape(8, 1, 128),
                 8, 4, 1, "relu_bias")

    Wgn = jnp.zeros((512, 128), f32)
    Wgn = Wgn.at[:, :3].set(Wg).at[:, 3:4].set(Wn)
    bgn = jnp.zeros((128,), f32).at[:3].set(bg).at[3:4].set(bn)

    def fin_body(xin, brow, w, bglob, y):
        accum = None
        for k in range(4):
            a = xin[k] + brow[k, 0][None, :]
            p = jnp.dot(a, w[k], preferred_element_type=f32)
            accum = p if accum is None else accum + p
        y[...] = accum + bglob[0][None, :]

    BN = 1024
    y = pl.pallas_call(
        fin_body,
        grid=(NPAD // BN,),
        in_specs=[
            pl.BlockSpec((4, BN, 128), lambda nb: (0, nb, 0)),
            pl.BlockSpec((4, 1, 128), lambda nb: (0, 0, 0)),
            pl.BlockSpec((4, 128, 128), lambda nb: (0, 0, 0)),
            pl.BlockSpec((1, 128), lambda nb: (0, 0)),
        ],
        out_specs=pl.BlockSpec((BN, 128), lambda nb: (nb, 0)),
        out_shape=jax.ShapeDtypeStruct((NPAD, 128), f32),
    )(out3, b3.reshape(4, 1, 128), Wgn.reshape(4, 128, 128), bgn.reshape(1, 128))

    return (y[:N, :3], y[:N, 3:4])
